# baseline (speedup 1.0000x reference)
import functools
import jax
import jax.numpy as jnp
from jax.experimental import pallas as pl
from jax.experimental.pallas import tpu as pltpu

E = 8
TOPK = 2
NEG_INF = -1e30


def _router_body(x_ref, wg_ref, logits_ref, coeff_ref):
    x = x_ref[...]
    wg = wg_ref[...]
    logits = jnp.dot(x, wg, preferred_element_type=jnp.float32)
    logits_ref[...] = logits
    L = logits.shape[0]
    iota = jax.lax.broadcasted_iota(jnp.int32, (L, E), 1)
    m1 = jnp.max(logits, axis=1, keepdims=True)
    a1 = jnp.min(jnp.where(logits == m1, iota, E), axis=1, keepdims=True)
    masked = jnp.where(iota == a1, NEG_INF, logits)
    m2 = jnp.max(masked, axis=1, keepdims=True)
    a2 = jnp.min(jnp.where(masked == m2, iota, E), axis=1, keepdims=True)
    w1 = 1.0 / (1.0 + jnp.exp(m2 - m1))
    w2 = 1.0 - w1
    coeff_ref[...] = jnp.where(iota == a1, w1, 0.0) + jnp.where(iota == a2, w2, 0.0)


def _expert_body(x_ref, w1_ref, w3_ref, w2_ref, coeff_ref, out_ref):
    e = pl.program_id(0)
    f = pl.program_id(1)
    x = x_ref[...]
    L = x.shape[0]
    iota = jax.lax.broadcasted_iota(jnp.int32, (L, E), 1)
    col = jnp.sum(jnp.where(iota == e, coeff_ref[...], 0.0), axis=1,
                  keepdims=True)

    @pl.when((e == 0) & (f == 0))
    def _():
        out_ref[...] = jnp.zeros_like(out_ref)

    g = jnp.dot(x, w1_ref[0], preferred_element_type=jnp.float32)
    u = jnp.dot(x, w3_ref[0], preferred_element_type=jnp.float32)
    h = ((g * jax.nn.sigmoid(g)) * u).astype(w2_ref.dtype)
    out_ref[...] += jnp.dot(h, w2_ref[0], preferred_element_type=jnp.float32) * col


def kernel(x, Wg, W1, W3, W2):
    Bs, L, D = x.shape
    x_flat = x.reshape(L, D)
    F = W1.shape[2]

    logits, coeff = pl.pallas_call(
        _router_body,
        out_shape=(
            jax.ShapeDtypeStruct((L, E), jnp.float32),
            jax.ShapeDtypeStruct((L, E), jnp.float32),
        ),
    )(x_flat, Wg)

    FC = 512
    xb = x_flat.astype(jnp.bfloat16)
    W1b = W1.astype(jnp.bfloat16)
    W3b = W3.astype(jnp.bfloat16)
    W2b = W2.astype(jnp.bfloat16)
    out = pl.pallas_call(
        _expert_body,
        grid=(E, F // FC),
        in_specs=[
            pl.BlockSpec((L, D), lambda e, f: (0, 0)),
            pl.BlockSpec((1, D, FC), lambda e, f: (e, 0, f)),
            pl.BlockSpec((1, D, FC), lambda e, f: (e, 0, f)),
            pl.BlockSpec((1, FC, D), lambda e, f: (e, f, 0)),
            pl.BlockSpec((L, E), lambda e, f: (0, 0)),
        ],
        out_specs=pl.BlockSpec((L, D), lambda e, f: (0, 0)),
        out_shape=jax.ShapeDtypeStruct((L, D), jnp.float32),
        compiler_params=pltpu.CompilerParams(
            dimension_semantics=("arbitrary", "arbitrary"),
        ),
    )(xb, W1b, W3b, W2b, coeff)

    return out.reshape(Bs, L, D), logits


# speedup vs baseline: 1.2481x; 1.2481x over previous
import functools
import jax
import jax.numpy as jnp
from jax import lax
from jax.experimental import pallas as pl
from jax.experimental.pallas import tpu as pltpu
from jax.experimental.pallas import tpu_sc as plsc

E = 8
NEG_INF = -1e30
BLK = 128
NB = 40
NBPAD = 48
NTILE = 32


def _router_body(x_ref, wg_ref, logits_ref, epair_ref, wpair_ref):
    x = x_ref[...]
    wg = wg_ref[...]
    logits = jnp.dot(x, wg, preferred_element_type=jnp.float32)
    logits_ref[...] = logits
    L = logits.shape[0]
    iota = lax.broadcasted_iota(jnp.int32, (L, E), 1)
    m1 = jnp.max(logits, axis=1, keepdims=True)
    a1 = jnp.min(jnp.where(logits == m1, iota, E), axis=1, keepdims=True)
    masked = jnp.where(iota == a1, NEG_INF, logits)
    m2 = jnp.max(masked, axis=1, keepdims=True)
    a2 = jnp.min(jnp.where(masked == m2, iota, E), axis=1, keepdims=True)
    w1 = 1.0 / (1.0 + jnp.exp(m2 - m1))
    epair_ref[...] = jnp.concatenate([a1, a2], axis=1)
    wpair_ref[...] = jnp.concatenate([w1, 1.0 - w1], axis=1)


def _splat(v):
    return jnp.broadcast_to(v, (16,)).astype(jnp.int32)


def _route_body(epair, pos_out, tok_out, bmap_out,
                ep_v, hist_v, sh_hist, allh_v, cnt_v, pos_v,
                posA_v, posB_v, tokA_v, tokB_v, bmap_v, sem):
    c = lax.axis_index("c")
    s = lax.axis_index("s")
    NP = pos_out.shape[0]
    CH = NP // 16
    NV = CH // 16
    lane = lax.broadcasted_iota(jnp.int32, (16,), 0)
    zero = jnp.zeros((16,), jnp.int32)
    one = jnp.ones((16,), jnp.int32)

    @pl.when(c == 0)
    def _phase_a():
        pltpu.sync_copy(epair.at[pl.ds(s * CH, CH)], ep_v)
        acc = zero
        for k in range(NV):
            e = ep_v[pl.ds(k * 16, 16)]
            for ee in range(E):
                m = e == _splat(ee)
                pc = _splat(jnp.max(plsc.cumsum(jnp.where(m, one, zero))))
                acc = acc + jnp.where(lane == _splat(ee), pc, zero)
        hist_v[...] = acc
        pltpu.sync_copy(hist_v, sh_hist.at[pl.ds(s * 16, 16)])

    plsc.subcore_barrier()

    @pl.when(c == 0)
    def _phase_bcd():
        pltpu.sync_copy(sh_hist, allh_v)
        pre = zero
        tot = zero
        for t in range(16):
            row = allh_v[pl.ds(t * 16, 16)]
            tot = tot + row
            pre = pre + jnp.where(_splat(s) > _splat(t), row, zero)
        padded = ((tot + _splat(BLK - 1)) >> _splat(7)) << _splat(7)
        ends = plsc.cumsum(padded)
        base = (ends - padded) + pre
        cnt_v[...] = base

        for k in range(NV):
            e = ep_v[pl.ds(k * 16, 16)]
            basg = plsc.load_gather(cnt_v, [e])
            rank = zero
            hch = zero
            for ee in range(E):
                m = e == _splat(ee)
                r = plsc.cumsum(jnp.where(m, one, zero))
                rank = rank + jnp.where(m, r - one, zero)
                hch = hch + jnp.where(lane == _splat(ee), _splat(jnp.max(r)), zero)
            pos = basg + rank
            pos_v[pl.ds(k * 16, 16)] = pos
            posH_v = posA_v if k < 8 else posB_v
            tokH_v = tokA_v if k < 8 else tokB_v
            posH_v[pl.ds((k % 8) * 16, 16)] = jnp.minimum(
                jnp.maximum(pos, _splat(0)), _splat(tok_out.shape[0] - 1))
            tokH_v[pl.ds((k % 8) * 16, 16)] = (
                _splat(s * CH + k * 16) + lane) >> _splat(1)
            cnt_v[...] = cnt_v[...] + hch

        pltpu.sync_copy(pos_v, pos_out.at[pl.ds(s * CH, CH)])
        pltpu.async_copy(tokA_v, tok_out.at[posA_v], sem).wait()
        pltpu.async_copy(tokB_v, tok_out.at[posB_v], sem).wait()

        @pl.when(s == 0)
        def _phase_d():
            for cidx in range(NBPAD // 16):
                sstart = (lane + _splat(cidx * 16)) << _splat(7)
                acc8 = zero
                for ee in range(E):
                    end_s = _splat(jnp.max(jnp.where(lane == _splat(ee), ends, zero)))
                    acc8 = acc8 + jnp.where(sstart >= end_s, one, zero)
                bmap_v[pl.ds(cidx * 16, 16)] = jnp.minimum(acc8, _splat(E - 1))
            pltpu.sync_copy(bmap_v, bmap_out)


def _gather_body(table, idx, out, idx_v, rows_v, sem, *, nrows):
    c = lax.axis_index("c")
    s = lax.axis_index("s")
    wid = c * 16 + s
    M = out.shape[0]
    N = table.shape[0]
    per = M // NTILE
    nch = per // nrows
    for ch in range(nch):
        off = wid * per + ch * nrows
        pltpu.sync_copy(idx.at[pl.ds(off, nrows)], idx_v)
        for k in range(nrows // 16):
            v = idx_v[pl.ds(k * 16, 16)]
            idx_v[pl.ds(k * 16, 16)] = jnp.minimum(
                jnp.maximum(v, _splat(0)), _splat(N - 1))
        pltpu.async_copy(table.at[idx_v], rows_v, sem).wait()
        pltpu.sync_copy(rows_v, out.at[pl.ds(off, nrows)])


def _gmm_body(bmap_ref, xs_ref, w1_ref, w3_ref, w2_ref, y_ref):
    xs = xs_ref[...]
    g = jnp.dot(xs, w1_ref[0], preferred_element_type=jnp.float32)
    u = jnp.dot(xs, w3_ref[0], preferred_element_type=jnp.float32)
    h = (g * jax.nn.sigmoid(g)) * u
    y_ref[...] = jnp.dot(h, w2_ref[0], preferred_element_type=jnp.float32)


def _combine_body(yp_ref, wpair_ref, out_ref):
    w0 = wpair_ref[:, 0:1]
    w1 = wpair_ref[:, 1:2]
    out_ref[...] = yp_ref[:, 0, :] * w0 + yp_ref[:, 1, :] * w1


def _sc_gather(table, idx, out_shape, nrows):
    D = table.shape[1]
    mesh = plsc.VectorSubcoreMesh(core_axis_name="c", subcore_axis_name="s")
    fn = functools.partial(
        pl.kernel,
        mesh=mesh,
        out_type=jax.ShapeDtypeStruct(out_shape, table.dtype),
        scratch_types=[
            pltpu.VMEM((nrows,), jnp.int32),
            pltpu.VMEM((nrows, D), table.dtype),
            pltpu.SemaphoreType.DMA,
        ],
        compiler_params=pltpu.CompilerParams(needs_layout_passes=False),
    )(functools.partial(_gather_body, nrows=nrows))
    return fn(table, idx)


def kernel(x, Wg, W1, W3, W2):
    Bs, L, D = x.shape
    x_flat = x.reshape(L, D)
    F = W1.shape[2]
    NP = 2 * L
    P = NB * BLK

    logits, epair, wpair = pl.pallas_call(
        _router_body,
        out_shape=(
            jax.ShapeDtypeStruct((L, E), jnp.float32),
            jax.ShapeDtypeStruct((L, 2), jnp.int32),
            jax.ShapeDtypeStruct((L, 2), jnp.float32),
        ),
    )(x_flat, Wg)

    mesh = plsc.VectorSubcoreMesh(core_axis_name="c", subcore_axis_name="s")
    pos_pair, tok_slot, bmap = functools.partial(
        pl.kernel,
        mesh=mesh,
        out_type=(
            jax.ShapeDtypeStruct((NP,), jnp.int32),
            jax.ShapeDtypeStruct((P,), jnp.int32),
            jax.ShapeDtypeStruct((NBPAD,), jnp.int32),
        ),
        scratch_types=[
            pltpu.VMEM((NP // 16,), jnp.int32),
            pltpu.VMEM((16,), jnp.int32),
            pltpu.VMEM_SHARED((256,), jnp.int32),
            pltpu.VMEM((256,), jnp.int32),
            pltpu.VMEM((16,), jnp.int32),
            pltpu.VMEM((NP // 16,), jnp.int32),
            pltpu.VMEM((128,), jnp.int32),
            pltpu.VMEM((128,), jnp.int32),
            pltpu.VMEM((128,), jnp.int32),
            pltpu.VMEM((128,), jnp.int32),
            pltpu.VMEM((NBPAD,), jnp.int32),
            pltpu.SemaphoreType.DMA,
        ],
        compiler_params=pltpu.CompilerParams(needs_layout_passes=False),
    )(_route_body)(epair.reshape(NP))

    xs = _sc_gather(x_flat, tok_slot, (P, D), 32)

    grid_spec = pltpu.PrefetchScalarGridSpec(
        num_scalar_prefetch=1,
        grid=(NB,),
        in_specs=[
            pl.BlockSpec((BLK, D), lambda b, m: (b, 0)),
            pl.BlockSpec((1, D, F), lambda b, m: (m[b], 0, 0)),
            pl.BlockSpec((1, D, F), lambda b, m: (m[b], 0, 0)),
            pl.BlockSpec((1, F, D), lambda b, m: (m[b], 0, 0)),
        ],
        out_specs=pl.BlockSpec((BLK, D), lambda b, m: (b, 0)),
    )
    y = pl.pallas_call(
        _gmm_body,
        grid_spec=grid_spec,
        out_shape=jax.ShapeDtypeStruct((P, D), jnp.float32),
        compiler_params=pltpu.CompilerParams(
            dimension_semantics=("arbitrary",),
        ),
    )(bmap, xs, W1, W3, W2)

    ypair = _sc_gather(y, pos_pair, (NP, D), 32)

    out = pl.pallas_call(
        _combine_body,
        out_shape=jax.ShapeDtypeStruct((L, D), jnp.float32),
    )(ypair.reshape(L, 2, D), wpair)

    return out.reshape(Bs, L, D), logits
